# baseline (device time: 33183 ns/iter reference)
import functools

import numpy as np

import jax
import jax.numpy as jnp
from jax import lax
from jax.experimental import pallas as pl
from jax.experimental.pallas import tpu as pltpu

N_DEV = 4
B, Sq, Skv, Hq, Dh = 2, 256, 256, 16, 64
H_LOC = Hq // N_DEV

_qb = (np.arange(Sq) // 64)[:, None]
_kb = (np.arange(Skv) // 64)[None, :]
_MASK = (_qb == _kb) | (_kb == 0) | ((_qb + _kb) % 3 == 0)


def _local_partial(x, Wq, K_ext, V_ext, Wo):
    my = lax.axis_index("i")
    bf = jnp.bfloat16
    xb = x.astype(bf)
    Q = (xb.reshape(B * Sq, -1) @ Wq.astype(bf)).reshape(B, Sq, H_LOC, Dh)
    K = lax.dynamic_slice_in_dim(K_ext, my * H_LOC, H_LOC, axis=2).astype(bf)
    V = lax.dynamic_slice_in_dim(V_ext, my * H_LOC, H_LOC, axis=2).astype(bf)
    scores = jnp.einsum(
        "bihd,bjhd->bhij", Q, K, preferred_element_type=jnp.float32
    ) * 0.125
    mask = jnp.asarray(_MASK)[None, None, :, :]
    scores = jnp.where(mask, scores, -1e9)
    w = jax.nn.softmax(scores, axis=-1)
    ctx = jnp.einsum(
        "bhij,bjhd->bihd", w.astype(bf), V, preferred_element_type=jnp.float32
    ).reshape(B, Sq, H_LOC * Dh)
    return (ctx.astype(bf).reshape(B * Sq, -1) @ Wo.astype(bf)).astype(bf)


def _ring_allreduce(partial2d):
    m, n = partial2d.shape

    def body(p_ref, out_ref, comm_ref, send_sems, recv_sems):
        my = lax.axis_index("i")
        left = lax.rem(my - 1 + N_DEV, N_DEV)
        right = lax.rem(my + 1, N_DEV)

        barrier_sem = pltpu.get_barrier_semaphore()
        for nbr in (left, right):
            pl.semaphore_signal(
                barrier_sem, inc=1,
                device_id=(nbr,), device_id_type=pl.DeviceIdType.MESH,
            )
        pl.semaphore_wait(barrier_sem, 2)

        comm_ref[0, :, :] = p_ref[:, :]
        out_ref[:, :] = p_ref[:, :].astype(jnp.float32)

        for h in range(N_DEV - 1):
            send_slot = h % 2
            recv_slot = (h + 1) % 2
            rdma = pltpu.make_async_remote_copy(
                src_ref=comm_ref.at[send_slot],
                dst_ref=comm_ref.at[recv_slot],
                send_sem=send_sems.at[send_slot],
                recv_sem=recv_sems.at[recv_slot],
                device_id=(right,),
                device_id_type=pl.DeviceIdType.MESH,
            )
            rdma.start()
            rdma.wait()
            out_ref[:, :] = out_ref[:, :] + comm_ref[recv_slot, :, :].astype(
                jnp.float32
            )

    return pl.pallas_call(
        body,
        out_shape=jax.ShapeDtypeStruct((m, n), jnp.float32),
        in_specs=[pl.BlockSpec(memory_space=pltpu.VMEM)],
        out_specs=pl.BlockSpec(memory_space=pltpu.VMEM),
        scratch_shapes=[
            pltpu.VMEM((2, m, n), jnp.bfloat16),
            pltpu.SemaphoreType.DMA((2,)),
            pltpu.SemaphoreType.DMA((2,)),
        ],
        compiler_params=pltpu.CompilerParams(collective_id=0),
    )(partial2d)


def kernel(x, Wq, K_ext, V_ext, Wo):
    partial = _local_partial(x, Wq, K_ext, V_ext, Wo)
    out = _ring_allreduce(partial)
    return out.reshape(B, Sq, -1)


# device time: 20625 ns/iter; 1.6089x vs baseline; 1.6089x over previous
import functools

import numpy as np

import jax
import jax.numpy as jnp
from jax import lax
from jax.experimental import pallas as pl
from jax.experimental.pallas import tpu as pltpu

N_DEV = 4
B, Sq, Skv, Hq, Dh = 2, 256, 256, 16, 64
H_LOC = Hq // N_DEV

_qb = (np.arange(Sq) // 64)[:, None]
_kb = (np.arange(Skv) // 64)[None, :]
_MASK = (_qb == _kb) | (_kb == 0) | ((_qb + _kb) % 3 == 0)


def _local_partial(x, Wq, K_ext, V_ext, Wo):
    my = lax.axis_index("i")
    bf = jnp.bfloat16
    xb = x.astype(bf)
    Q = (xb.reshape(B * Sq, -1) @ Wq.astype(bf)).reshape(B, Sq, H_LOC, Dh)
    K = lax.dynamic_slice_in_dim(K_ext, my * H_LOC, H_LOC, axis=2).astype(bf)
    V = lax.dynamic_slice_in_dim(V_ext, my * H_LOC, H_LOC, axis=2).astype(bf)
    scores = jnp.einsum(
        "bihd,bjhd->bhij", Q, K, preferred_element_type=jnp.float32
    ) * 0.125
    mask = jnp.asarray(_MASK)[None, None, :, :]
    scores = jnp.where(mask, scores, -1e9)
    w = jax.nn.softmax(scores, axis=-1)
    ctx = jnp.einsum(
        "bhij,bjhd->bihd", w.astype(bf), V, preferred_element_type=jnp.float32
    ).reshape(B, Sq, H_LOC * Dh)
    return (ctx.astype(bf).reshape(B * Sq, -1) @ Wo.astype(bf)).astype(bf)


def _allreduce_a2a(partial2d):
    m, n = partial2d.shape
    c = m // N_DEV

    def body(p_ref, out_ref, red_ref, rs_buf, ag_buf,
             rs_send, rs_recv, ag_send, ag_recv):
        my = lax.axis_index("i")

        barrier_sem = pltpu.get_barrier_semaphore()
        for j in range(1, N_DEV):
            pl.semaphore_signal(
                barrier_sem, inc=1,
                device_id=(lax.rem(my + j, N_DEV),),
                device_id_type=pl.DeviceIdType.MESH,
            )
        pl.semaphore_wait(barrier_sem, N_DEV - 1)

        rs_by_slot = [None] * (N_DEV - 1)
        for j in range(1, N_DEV):
            dst = lax.rem(my + j, N_DEV)
            slot = N_DEV - 1 - j
            rdma = pltpu.make_async_remote_copy(
                src_ref=p_ref.at[pl.ds(dst * c, c)],
                dst_ref=rs_buf.at[slot],
                send_sem=rs_send.at[j - 1],
                recv_sem=rs_recv.at[slot],
                device_id=(dst,),
                device_id_type=pl.DeviceIdType.MESH,
            )
            rdma.start()
            rs_by_slot[slot] = rdma

        acc = p_ref[pl.ds(my * c, c), :].astype(jnp.float32)
        for k in range(N_DEV - 1):
            rs_by_slot[k].wait_recv()
            acc = acc + rs_buf[k, :, :].astype(jnp.float32)
        red_ref[:, :] = acc.astype(jnp.bfloat16)
        out_ref[pl.ds(my * c, c), :] = acc

        ag_by_slot = [None] * (N_DEV - 1)
        for j in range(1, N_DEV):
            dst = lax.rem(my + j, N_DEV)
            slot = N_DEV - 1 - j
            rdma = pltpu.make_async_remote_copy(
                src_ref=red_ref,
                dst_ref=ag_buf.at[slot],
                send_sem=ag_send.at[j - 1],
                recv_sem=ag_recv.at[slot],
                device_id=(dst,),
                device_id_type=pl.DeviceIdType.MESH,
            )
            rdma.start()
            ag_by_slot[slot] = rdma

        for k in range(N_DEV - 1):
            ag_by_slot[k].wait_recv()
            src_dev = lax.rem(my + 1 + k, N_DEV)
            out_ref[pl.ds(src_dev * c, c), :] = ag_buf[k, :, :].astype(
                jnp.float32
            )

        for rdma in rs_by_slot + ag_by_slot:
            rdma.wait_send()

    return pl.pallas_call(
        body,
        out_shape=jax.ShapeDtypeStruct((m, n), jnp.float32),
        in_specs=[pl.BlockSpec(memory_space=pltpu.VMEM)],
        out_specs=pl.BlockSpec(memory_space=pltpu.VMEM),
        scratch_shapes=[
            pltpu.VMEM((c, n), jnp.bfloat16),
            pltpu.VMEM((N_DEV - 1, c, n), jnp.bfloat16),
            pltpu.VMEM((N_DEV - 1, c, n), jnp.bfloat16),
            pltpu.SemaphoreType.DMA((N_DEV - 1,)),
            pltpu.SemaphoreType.DMA((N_DEV - 1,)),
            pltpu.SemaphoreType.DMA((N_DEV - 1,)),
            pltpu.SemaphoreType.DMA((N_DEV - 1,)),
        ],
        compiler_params=pltpu.CompilerParams(collective_id=0),
    )(partial2d)


def kernel(x, Wq, K_ext, V_ext, Wo):
    partial = _local_partial(x, Wq, K_ext, V_ext, Wo)
    out = _allreduce_a2a(partial)
    return out.reshape(B, Sq, -1)


# device time: 19573 ns/iter; 1.6953x vs baseline; 1.0537x over previous
import jax
import jax.numpy as jnp
from jax import lax
from jax.experimental import pallas as pl
from jax.experimental.pallas import tpu as pltpu

N_DEV = 4
B, Sq, Skv, Hq, Dh = 2, 256, 256, 16, 64
H_LOC = Hq // N_DEV
M, N = B * Sq, 512
C = M // N_DEV

_bf = jnp.bfloat16
_f32 = jnp.float32


def _chunk_mask(qo):
    qb = (qo + lax.broadcasted_iota(jnp.int32, (C, Skv), 0)) // 64
    kb = lax.broadcasted_iota(jnp.int32, (C, Skv), 1) // 64
    return (qb == kb) | (kb == 0) | (lax.rem(qb + kb, 3) == 0)


def _fused(x2, Wq, Kl, Vl, Wo):
    def body(x_ref, wq_ref, k_ref, v_ref, wo_ref, out_ref,
             chunks, rs_buf, ag_buf, red_ref,
             rs_send, rs_recv, ag_send, ag_recv):
        my = lax.axis_index("i")

        barrier_sem = pltpu.get_barrier_semaphore()
        for j in range(1, N_DEV):
            pl.semaphore_signal(
                barrier_sem, inc=1,
                device_id=(lax.rem(my + j, N_DEV),),
                device_id_type=pl.DeviceIdType.MESH,
            )

        def compute_chunk(j):
            b, qo = j // 2, (j % 2) * C
            xc = x_ref[pl.ds(j * C, C), :]
            qc = jnp.dot(xc, wq_ref[:, :],
                         preferred_element_type=_f32).astype(_bf)
            mask = _chunk_mask(qo)
            ctx_parts = []
            for h in range(H_LOC):
                q = qc[:, h * Dh:(h + 1) * Dh]
                kh = k_ref[b, h, :, :]
                s = lax.dot_general(
                    q, kh, (((1,), (1,)), ((), ())),
                    preferred_element_type=_f32,
                ) * 0.125
                s = jnp.where(mask, s, -1e9)
                w = jnp.exp(s - jnp.max(s, axis=-1, keepdims=True))
                w = w / jnp.sum(w, axis=-1, keepdims=True)
                ctx_parts.append(
                    jnp.dot(w.astype(_bf), v_ref[b, h, :, :],
                            preferred_element_type=_f32).astype(_bf)
                )
            ctx = jnp.concatenate(ctx_parts, axis=1)
            chunks[j, :, :] = jnp.dot(
                ctx, wo_ref[:, :], preferred_element_type=_f32
            ).astype(_bf)

        def rs_send_rdma(j):
            return pltpu.make_async_remote_copy(
                src_ref=chunks.at[j],
                dst_ref=rs_buf.at[my],
                send_sem=rs_send.at[j],
                recv_sem=rs_recv.at[my],
                device_id=(j,),
                device_id_type=pl.DeviceIdType.MESH,
            )

        rs_rdmas = [rs_send_rdma(j) for j in range(N_DEV)]

        compute_chunk(0)
        pl.semaphore_wait(barrier_sem, N_DEV - 1)
        pl.when(my != 0)(rs_rdmas[0].start)
        for j in range(1, N_DEV):
            compute_chunk(j)
            pl.when(my != j)(rs_rdmas[j].start)

        def rs_recv_rdma(s):
            return pltpu.make_async_remote_copy(
                src_ref=chunks.at[s], dst_ref=rs_buf.at[s],
                send_sem=rs_send.at[s], recv_sem=rs_recv.at[s],
                device_id=(s,), device_id_type=pl.DeviceIdType.MESH,
            )

        acc = chunks[my, :, :].astype(_f32)
        for s in range(N_DEV):
            pl.when(my != s)(rs_recv_rdma(s).wait_recv)
            acc = acc + jnp.where(
                my == s, 0.0, rs_buf[s, :, :].astype(_f32)
            )
        red_ref[:, :] = acc.astype(_bf)
        out_ref[pl.ds(my * C, C), :] = red_ref[:, :]

        def ag_send_rdma(j):
            return pltpu.make_async_remote_copy(
                src_ref=red_ref,
                dst_ref=ag_buf.at[my],
                send_sem=ag_send.at[j],
                recv_sem=ag_recv.at[my],
                device_id=(j,),
                device_id_type=pl.DeviceIdType.MESH,
            )

        def ag_recv_rdma(s):
            return pltpu.make_async_remote_copy(
                src_ref=red_ref, dst_ref=ag_buf.at[s],
                send_sem=ag_send.at[s], recv_sem=ag_recv.at[s],
                device_id=(s,), device_id_type=pl.DeviceIdType.MESH,
            )

        ag_rdmas = [ag_send_rdma(j) for j in range(N_DEV)]
        for j in range(N_DEV):
            pl.when(my != j)(ag_rdmas[j].start)

        for s in range(N_DEV):
            @pl.when(my != s)
            def _(s=s):
                ag_recv_rdma(s).wait_recv()
                out_ref[pl.ds(s * C, C), :] = ag_buf[s, :, :]

        for j in range(N_DEV):
            pl.when(my != j)(rs_rdmas[j].wait_send)
            pl.when(my != j)(ag_rdmas[j].wait_send)

    return pl.pallas_call(
        body,
        out_shape=jax.ShapeDtypeStruct((M, N), _bf),
        in_specs=[pl.BlockSpec(memory_space=pltpu.VMEM)] * 5,
        out_specs=pl.BlockSpec(memory_space=pltpu.VMEM),
        scratch_shapes=[
            pltpu.VMEM((N_DEV, C, N), _bf),
            pltpu.VMEM((N_DEV, C, N), _bf),
            pltpu.VMEM((N_DEV, C, N), _bf),
            pltpu.VMEM((C, N), _bf),
            pltpu.SemaphoreType.DMA((N_DEV,)),
            pltpu.SemaphoreType.DMA((N_DEV,)),
            pltpu.SemaphoreType.DMA((N_DEV,)),
            pltpu.SemaphoreType.DMA((N_DEV,)),
        ],
        compiler_params=pltpu.CompilerParams(collective_id=0),
    )(x2, Wq, Kl, Vl, Wo)


def kernel(x, Wq, K_ext, V_ext, Wo):
    my = lax.axis_index("i")
    x2 = x.reshape(M, -1).astype(_bf)
    Kl = lax.dynamic_slice_in_dim(K_ext, my * H_LOC, H_LOC, axis=2)
    Vl = lax.dynamic_slice_in_dim(V_ext, my * H_LOC, H_LOC, axis=2)
    Kl = Kl.astype(_bf).transpose(0, 2, 1, 3)
    Vl = Vl.astype(_bf).transpose(0, 2, 1, 3)
    out = _fused(x2, Wq.astype(_bf), Kl, Vl, Wo.astype(_bf))
    return out.reshape(B, Sq, -1)


# device time: 16610 ns/iter; 1.9978x vs baseline; 1.1784x over previous
import jax
import jax.numpy as jnp
from jax import lax
from jax.experimental import pallas as pl
from jax.experimental.pallas import tpu as pltpu

N_DEV = 4
B, Sq, Skv, Hq, Dh = 2, 256, 256, 16, 64
H_LOC = Hq // N_DEV
M, N = B * Sq, 512
C = M // N_DEV

_bf = jnp.bfloat16
_f32 = jnp.float32


def _fused(x2, Wq, Kl, Vl, Wo):
    def body(x_ref, wq_ref, k_ref, v_ref, wo_ref, out_ref,
             chunks, rs_buf, red_ref, wq_bf, wo_bf,
             rs_send, rs_recv, ag_send, ag_recv):
        my = lax.axis_index("i")

        barrier_sem = pltpu.get_barrier_semaphore()
        for j in range(1, N_DEV):
            pl.semaphore_signal(
                barrier_sem, inc=1,
                device_id=(lax.rem(my + j, N_DEV),),
                device_id_type=pl.DeviceIdType.MESH,
            )

        wq_bf[:, :] = wq_ref[:, :].astype(_bf)
        wo_bf[:, :] = wo_ref[:, :].astype(_bf)

        def compute_chunk(j):
            b = lax.div(j, 2)
            qo = lax.rem(j, 2) * C
            xc = x_ref[pl.ds(j * C, C), :].astype(_bf)
            qc = jnp.dot(xc, wq_bf[:, :],
                         preferred_element_type=_f32).astype(_bf)
            qb = (qo + lax.broadcasted_iota(jnp.int32, (C, Skv), 0)) // 64
            kb = lax.broadcasted_iota(jnp.int32, (C, Skv), 1) // 64
            mask = (qb == kb) | (kb == 0) | (lax.rem(qb + kb, 3) == 0)
            ctx_parts = []
            for h in range(H_LOC):
                q = qc[:, h * Dh:(h + 1) * Dh]
                kh = k_ref[b * H_LOC + h, :, :]
                s = lax.dot_general(
                    q, kh, (((1,), (1,)), ((), ())),
                    preferred_element_type=_f32,
                ) * 0.125
                s = jnp.where(mask, s, -1e9)
                w = jnp.exp(s - jnp.max(s, axis=-1, keepdims=True))
                w = w / jnp.sum(w, axis=-1, keepdims=True)
                ctx_parts.append(
                    jnp.dot(w.astype(_bf), v_ref[b * H_LOC + h, :, :],
                            preferred_element_type=_f32).astype(_bf)
                )
            ctx = jnp.concatenate(ctx_parts, axis=1)
            return jnp.dot(ctx, wo_bf[:, :], preferred_element_type=_f32)

        send_descs = []
        for t in range(N_DEV - 1):
            dst = lax.rem(my + 1 + t, N_DEV)
            pc = compute_chunk(dst)
            chunks[dst, :, :] = pc.astype(_bf)
            if t == 0:
                pl.semaphore_wait(barrier_sem, N_DEV - 1)
            rdma = pltpu.make_async_remote_copy(
                src_ref=chunks.at[dst],
                dst_ref=rs_buf.at[my],
                send_sem=rs_send.at[t],
                recv_sem=rs_recv.at[my],
                device_id=(dst,),
                device_id_type=pl.DeviceIdType.MESH,
            )
            rdma.start()
            send_descs.append(rdma)

        acc = compute_chunk(my)

        for s in range(N_DEV):
            recv = pltpu.make_async_remote_copy(
                src_ref=chunks.at[s], dst_ref=rs_buf.at[s],
                send_sem=rs_send.at[0], recv_sem=rs_recv.at[s],
                device_id=(s,), device_id_type=pl.DeviceIdType.MESH,
            )
            pl.when(my != s)(recv.wait_recv)
            acc = acc + jnp.where(my == s, 0.0, rs_buf[s, :, :].astype(_f32))
        red_ref[:, :] = acc.astype(_bf)
        out_ref[pl.ds(my * C, C), :] = red_ref[:, :]

        for t in range(N_DEV - 1):
            dst = lax.rem(my + 1 + t, N_DEV)
            rdma = pltpu.make_async_remote_copy(
                src_ref=red_ref,
                dst_ref=out_ref.at[pl.ds(my * C, C)],
                send_sem=ag_send.at[t],
                recv_sem=ag_recv.at[my],
                device_id=(dst,),
                device_id_type=pl.DeviceIdType.MESH,
            )
            rdma.start()
            send_descs.append(rdma)

        for s in range(N_DEV):
            recv = pltpu.make_async_remote_copy(
                src_ref=red_ref, dst_ref=out_ref.at[pl.ds(s * C, C)],
                send_sem=ag_send.at[0], recv_sem=ag_recv.at[s],
                device_id=(s,), device_id_type=pl.DeviceIdType.MESH,
            )
            pl.when(my != s)(recv.wait_recv)

        for rdma in send_descs:
            rdma.wait_send()

    return pl.pallas_call(
        body,
        out_shape=jax.ShapeDtypeStruct((M, N), _bf),
        in_specs=[pl.BlockSpec(memory_space=pltpu.VMEM)] * 5,
        out_specs=pl.BlockSpec(memory_space=pltpu.VMEM),
        scratch_shapes=[
            pltpu.VMEM((N_DEV, C, N), _bf),
            pltpu.VMEM((N_DEV, C, N), _bf),
            pltpu.VMEM((C, N), _bf),
            pltpu.VMEM((512, 256), _bf),
            pltpu.VMEM((256, 512), _bf),
            pltpu.SemaphoreType.DMA((N_DEV - 1,)),
            pltpu.SemaphoreType.DMA((N_DEV,)),
            pltpu.SemaphoreType.DMA((N_DEV - 1,)),
            pltpu.SemaphoreType.DMA((N_DEV,)),
        ],
        compiler_params=pltpu.CompilerParams(collective_id=0),
    )(x2, Wq, Kl, Vl, Wo)


def kernel(x, Wq, K_ext, V_ext, Wo):
    my = lax.axis_index("i")
    x2 = x.reshape(M, -1)
    Kl = lax.dynamic_slice_in_dim(K_ext, my * H_LOC, H_LOC, axis=2)
    Vl = lax.dynamic_slice_in_dim(V_ext, my * H_LOC, H_LOC, axis=2)
    Kl = Kl.astype(_bf).transpose(0, 2, 1, 3).reshape(B * H_LOC, Skv, Dh)
    Vl = Vl.astype(_bf).transpose(0, 2, 1, 3).reshape(B * H_LOC, Skv, Dh)
    out = _fused(x2, Wq, Kl, Vl, Wo)
    return out.reshape(B, Sq, -1)


# device time: 15906 ns/iter; 2.0862x vs baseline; 1.0443x over previous
import jax
import jax.numpy as jnp
from jax import lax
from jax.experimental import pallas as pl
from jax.experimental.pallas import tpu as pltpu

N_DEV = 4
B, Sq, Skv, Hq, Dh = 2, 256, 256, 16, 64
H_LOC = Hq // N_DEV
M, N = B * Sq, 512
C = M // N_DEV

_bf = jnp.bfloat16
_f32 = jnp.float32


def _fused(x2, Wq, Kl, Vl, Wo):
    def body(x_ref, wq_ref, k_ref, v_ref, wo_ref, out_ref,
             chunks, rs_buf, red_ref, wq_bf, wo_bf,
             rs_send, rs_recv, ag_send, ag_recv):
        my = lax.axis_index("i")

        barrier_sem = pltpu.get_barrier_semaphore()
        for j in range(1, N_DEV):
            pl.semaphore_signal(
                barrier_sem, inc=1,
                device_id=(lax.rem(my + j, N_DEV),),
                device_id_type=pl.DeviceIdType.MESH,
            )

        wq_bf[:, :] = (wq_ref[:, :] * 0.125).astype(_bf)
        wo_bf[:, :] = wo_ref[:, :].astype(_bf)

        def compute_chunk(j):
            b = lax.div(j, 2)
            qo = lax.rem(j, 2) * C
            xc = x_ref[pl.ds(j * C, C), :].astype(_bf)
            qc = jnp.dot(xc, wq_bf[:, :],
                         preferred_element_type=_f32).astype(_bf)
            qb = (qo + lax.broadcasted_iota(jnp.int32, (C, Skv), 0)) // 64
            kb = lax.broadcasted_iota(jnp.int32, (C, Skv), 1) // 64
            maskf = ((qb == kb) | (kb == 0) | (lax.rem(qb + kb, 3) == 0)
                     ).astype(_f32)
            ctx_parts = []
            for h in range(H_LOC):
                q = qc[:, h * Dh:(h + 1) * Dh]
                kh = k_ref[b * H_LOC + h, :, :]
                s = lax.dot_general(
                    q, kh, (((1,), (1,)), ((), ())),
                    preferred_element_type=_f32,
                )
                w = jnp.exp(s) * maskf
                w = w / jnp.sum(w, axis=-1, keepdims=True)
                ctx_parts.append(
                    jnp.dot(w.astype(_bf), v_ref[b * H_LOC + h, :, :],
                            preferred_element_type=_f32).astype(_bf)
                )
            ctx = jnp.concatenate(ctx_parts, axis=1)
            return jnp.dot(ctx, wo_bf[:, :], preferred_element_type=_f32)

        send_descs = []
        for t in range(N_DEV - 1):
            dst = lax.rem(my + 1 + t, N_DEV)
            pc = compute_chunk(dst)
            chunks[dst, :, :] = pc.astype(_bf)
            if t == 0:
                pl.semaphore_wait(barrier_sem, N_DEV - 1)
            rdma = pltpu.make_async_remote_copy(
                src_ref=chunks.at[dst],
                dst_ref=rs_buf.at[my],
                send_sem=rs_send.at[t],
                recv_sem=rs_recv.at[my],
                device_id=(dst,),
                device_id_type=pl.DeviceIdType.MESH,
            )
            rdma.start()
            send_descs.append(rdma)

        acc = compute_chunk(my)

        for s in range(N_DEV):
            recv = pltpu.make_async_remote_copy(
                src_ref=chunks.at[s], dst_ref=rs_buf.at[s],
                send_sem=rs_send.at[0], recv_sem=rs_recv.at[s],
                device_id=(s,), device_id_type=pl.DeviceIdType.MESH,
            )
            pl.when(my != s)(recv.wait_recv)
            acc = acc + jnp.where(my == s, 0.0, rs_buf[s, :, :].astype(_f32))
        red_ref[:, :] = acc.astype(_bf)
        out_ref[pl.ds(my * C, C), :] = red_ref[:, :]

        for t in range(N_DEV - 1):
            dst = lax.rem(my + 1 + t, N_DEV)
            rdma = pltpu.make_async_remote_copy(
                src_ref=red_ref,
                dst_ref=out_ref.at[pl.ds(my * C, C)],
                send_sem=ag_send.at[t],
                recv_sem=ag_recv.at[my],
                device_id=(dst,),
                device_id_type=pl.DeviceIdType.MESH,
            )
            rdma.start()
            send_descs.append(rdma)

        for s in range(N_DEV):
            recv = pltpu.make_async_remote_copy(
                src_ref=red_ref, dst_ref=out_ref.at[pl.ds(s * C, C)],
                send_sem=ag_send.at[0], recv_sem=ag_recv.at[s],
                device_id=(s,), device_id_type=pl.DeviceIdType.MESH,
            )
            pl.when(my != s)(recv.wait_recv)

        for rdma in send_descs:
            rdma.wait_send()

    return pl.pallas_call(
        body,
        out_shape=jax.ShapeDtypeStruct((M, N), _bf),
        in_specs=[pl.BlockSpec(memory_space=pltpu.VMEM)] * 5,
        out_specs=pl.BlockSpec(memory_space=pltpu.VMEM),
        scratch_shapes=[
            pltpu.VMEM((N_DEV, C, N), _bf),
            pltpu.VMEM((N_DEV, C, N), _bf),
            pltpu.VMEM((C, N), _bf),
            pltpu.VMEM((512, 256), _bf),
            pltpu.VMEM((256, 512), _bf),
            pltpu.SemaphoreType.DMA((N_DEV - 1,)),
            pltpu.SemaphoreType.DMA((N_DEV,)),
            pltpu.SemaphoreType.DMA((N_DEV - 1,)),
            pltpu.SemaphoreType.DMA((N_DEV,)),
        ],
        compiler_params=pltpu.CompilerParams(collective_id=0),
    )(x2, Wq, Kl, Vl, Wo)


def kernel(x, Wq, K_ext, V_ext, Wo):
    my = lax.axis_index("i")
    x2 = x.reshape(M, -1)
    Kl = lax.dynamic_slice_in_dim(K_ext, my * H_LOC, H_LOC, axis=2)
    Vl = lax.dynamic_slice_in_dim(V_ext, my * H_LOC, H_LOC, axis=2)
    Kl = Kl.astype(_bf).transpose(0, 2, 1, 3).reshape(B * H_LOC, Skv, Dh)
    Vl = Vl.astype(_bf).transpose(0, 2, 1, 3).reshape(B * H_LOC, Skv, Dh)
    out = _fused(x2, Wq, Kl, Vl, Wo)
    return out.reshape(B, Sq, -1)
